# Initial kernel scaffold; baseline (speedup 1.0000x reference)
import jax
import jax.numpy as jnp
from jax.experimental import pallas as pl


def kernel(hidden_states, gate_w, gate_b, expert_w, expert_b):
    raise NotImplementedError("write your pallas kernel here")


# speedup vs baseline: 2.4604x; 2.4604x over previous
import functools

import jax
import jax.numpy as jnp
from jax.experimental import pallas as pl
from jax.experimental.pallas import tpu as pltpu

_BT = 512


def _moe_body(x_ref, wg_ref, bg_ref, we_ref, be_ref, out_ref, *, E, C):
    x = x_ref[...]
    gate = (
        jax.lax.dot_general(
            x, wg_ref[...], (((1,), (0,)), ((), ())),
            preferred_element_type=jnp.float32,
        )
        + bg_ref[...]
    )
    eo = (
        jax.lax.dot_general(
            x, we_ref[...], (((1,), (0,)), ((), ())),
            preferred_element_type=jnp.float32,
        )
        + be_ref[...]
    )
    bt = gate.shape[0]
    ids = jax.lax.broadcasted_iota(jnp.int32, (bt, E), 1)
    m1 = jnp.max(gate, axis=1, keepdims=True)
    idx1 = jnp.min(jnp.where(gate == m1, ids, E), axis=1, keepdims=True)
    g2 = jnp.where(ids == idx1, -jnp.inf, gate)
    m2 = jnp.max(g2, axis=1, keepdims=True)
    idx2 = jnp.min(jnp.where(g2 == m2, ids, E), axis=1, keepdims=True)
    w1 = 1.0 / (1.0 + jnp.exp(m2 - m1))
    w2 = 1.0 - w1
    wts = jnp.where(ids == idx1, w1, 0.0) + jnp.where(ids == idx2, w2, 0.0)
    outs = [
        jnp.sum(wts * eo[:, c * E:(c + 1) * E], axis=1, keepdims=True)
        for c in range(C)
    ]
    out_ref[...] = jnp.concatenate(outs, axis=1)


def kernel(hidden_states, gate_w, gate_b, expert_w, expert_b):
    T, D = hidden_states.shape
    E = gate_w.shape[1]
    C = expert_w.shape[2]
    we = jnp.transpose(expert_w, (1, 2, 0)).reshape(D, C * E)
    be = jnp.transpose(expert_b, (1, 0)).reshape(1, C * E)
    bg = gate_b.reshape(1, E)
    grid = (T // _BT,)
    return pl.pallas_call(
        functools.partial(_moe_body, E=E, C=C),
        grid=grid,
        in_specs=[
            pl.BlockSpec((_BT, D), lambda i: (i, 0)),
            pl.BlockSpec((D, E), lambda i: (0, 0)),
            pl.BlockSpec((1, E), lambda i: (0, 0)),
            pl.BlockSpec((D, C * E), lambda i: (0, 0)),
            pl.BlockSpec((1, C * E), lambda i: (0, 0)),
        ],
        out_specs=pl.BlockSpec((_BT, 2), lambda i: (i, 0)),
        out_shape=jax.ShapeDtypeStruct((T, 2), jnp.float32),
        compiler_params=pltpu.CompilerParams(
            dimension_semantics=("arbitrary",),
        ),
    )(hidden_states, gate_w, bg, we, be)


# speedup vs baseline: 2.9001x; 1.1787x over previous
import functools

import jax
import jax.numpy as jnp
from jax.experimental import pallas as pl
from jax.experimental.pallas import tpu as pltpu

_BT = 512


def _moe_body(x_ref, w_ref, b_ref, out_ref, *, E, C):
    x = x_ref[...]
    y = (
        jax.lax.dot_general(
            x, w_ref[...], (((1,), (0,)), ((), ())),
            preferred_element_type=jnp.float32,
        )
        + b_ref[...]
    )
    gate = y[:, 0:E]
    bt = gate.shape[0]
    ids = jax.lax.broadcasted_iota(jnp.int32, (bt, E), 1)
    m1 = jnp.max(gate, axis=1, keepdims=True)
    idx1 = jnp.min(jnp.where(gate == m1, ids, E), axis=1, keepdims=True)
    g2 = jnp.where(ids == idx1, -jnp.inf, gate)
    m2 = jnp.max(g2, axis=1, keepdims=True)
    idx2 = jnp.min(jnp.where(g2 == m2, ids, E), axis=1, keepdims=True)
    w1 = 1.0 / (1.0 + jnp.exp(m2 - m1))
    w2 = 1.0 - w1
    wts = jnp.where(ids == idx1, w1, 0.0) + jnp.where(ids == idx2, w2, 0.0)
    outs = [
        jnp.sum(wts * y[:, (1 + c) * E:(2 + c) * E], axis=1, keepdims=True)
        for c in range(C)
    ]
    out_ref[...] = jnp.concatenate(outs, axis=1)


def kernel(hidden_states, gate_w, gate_b, expert_w, expert_b):
    T, D = hidden_states.shape
    E = gate_w.shape[1]
    C = expert_w.shape[2]
    we = jnp.transpose(expert_w, (1, 2, 0)).reshape(D, C * E)
    w = jnp.concatenate([gate_w, we], axis=1)
    b = jnp.concatenate(
        [gate_b.reshape(1, E), jnp.transpose(expert_b, (1, 0)).reshape(1, C * E)],
        axis=1,
    )
    grid = (T // _BT,)
    return pl.pallas_call(
        functools.partial(_moe_body, E=E, C=C),
        grid=grid,
        in_specs=[
            pl.BlockSpec((_BT, D), lambda i: (i, 0)),
            pl.BlockSpec((D, E + C * E), lambda i: (0, 0)),
            pl.BlockSpec((1, E + C * E), lambda i: (0, 0)),
        ],
        out_specs=pl.BlockSpec((_BT, 2), lambda i: (i, 0)),
        out_shape=jax.ShapeDtypeStruct((T, 2), jnp.float32),
        compiler_params=pltpu.CompilerParams(
            dimension_semantics=("arbitrary",),
        ),
    )(hidden_states, w, b)


# speedup vs baseline: 3.1933x; 1.1011x over previous
import functools

import jax
import jax.numpy as jnp
from jax.experimental import pallas as pl
from jax.experimental.pallas import tpu as pltpu

_BT = 1024


def _moe_body(x_ref, w_ref, b_ref, out_ref, *, E, C):
    x = x_ref[...]
    y = (
        jax.lax.dot_general(
            x, w_ref[...], (((1,), (0,)), ((), ())),
            preferred_element_type=jnp.float32,
        )
        + b_ref[...]
    )
    gate = y[:, 0:E]
    bt = gate.shape[0]
    ids = jax.lax.broadcasted_iota(jnp.int32, (bt, E), 1)
    m1 = jnp.max(gate, axis=1, keepdims=True)
    idx1 = jnp.min(jnp.where(gate == m1, ids, E), axis=1, keepdims=True)
    g2 = jnp.where(ids == idx1, -jnp.inf, gate)
    m2 = jnp.max(g2, axis=1, keepdims=True)
    idx2 = jnp.min(jnp.where(g2 == m2, ids, E), axis=1, keepdims=True)
    w1 = 1.0 / (1.0 + jnp.exp(m2 - m1))
    w2 = 1.0 - w1
    wts = jnp.where(ids == idx1, w1, 0.0) + jnp.where(ids == idx2, w2, 0.0)
    outs = [
        jnp.sum(wts * y[:, (1 + c) * E:(2 + c) * E], axis=1, keepdims=True)
        for c in range(C)
    ]
    out_ref[...] = jnp.concatenate(outs, axis=1)


def kernel(hidden_states, gate_w, gate_b, expert_w, expert_b):
    T, D = hidden_states.shape
    E = gate_w.shape[1]
    C = expert_w.shape[2]
    we = jnp.transpose(expert_w, (1, 2, 0)).reshape(D, C * E)
    w = jnp.concatenate([gate_w, we], axis=1)
    b = jnp.concatenate(
        [gate_b.reshape(1, E), jnp.transpose(expert_b, (1, 0)).reshape(1, C * E)],
        axis=1,
    )
    grid = (T // _BT,)
    return pl.pallas_call(
        functools.partial(_moe_body, E=E, C=C),
        grid=grid,
        in_specs=[
            pl.BlockSpec((_BT, D), lambda i: (i, 0)),
            pl.BlockSpec((D, E + C * E), lambda i: (0, 0)),
            pl.BlockSpec((1, E + C * E), lambda i: (0, 0)),
        ],
        out_specs=pl.BlockSpec((_BT, 2), lambda i: (i, 0)),
        out_shape=jax.ShapeDtypeStruct((T, 2), jnp.float32),
        compiler_params=pltpu.CompilerParams(
            dimension_semantics=("arbitrary",),
        ),
    )(hidden_states, w, b)
